# baseline (speedup 1.0000x reference)
import functools

import jax
import jax.numpy as jnp
from jax import lax
from jax.experimental import pallas as pl
from jax.experimental.pallas import tpu as pltpu
from jax.experimental.pallas import tpu_sc as plsc

D = 64
B_TOTAL = 4096 * 200

_info = plsc.get_sparse_core_info()
NC, NS = _info.num_cores, _info.num_subcores
NW = NC * NS
B_PER_W = B_TOTAL // NW
CHUNK = 512
N_CHUNKS = B_PER_W // CHUNK

_mesh = plsc.VectorSubcoreMesh(core_axis_name="c", subcore_axis_name="s")


@functools.partial(
    pl.kernel,
    mesh=_mesh,
    out_type=jax.ShapeDtypeStruct((B_TOTAL, D), jnp.float32),
    compiler_params=pltpu.CompilerParams(use_tc_tiling_on_sc=False),
    scratch_types=[
        pltpu.VMEM((CHUNK,), jnp.int32),
        pltpu.VMEM((CHUNK, D), jnp.float32),
        pltpu.SemaphoreType.DMA,
    ],
)
def _gather_kernel(idx_hbm, table_hbm, out_hbm, idx_v, rows_v, sem):
    wid = lax.axis_index("s") * NC + lax.axis_index("c")
    base = wid * B_PER_W

    def body(i, carry):
        off = base + i * CHUNK
        pltpu.sync_copy(idx_hbm.at[pl.ds(off, CHUNK)], idx_v)
        pltpu.async_copy(table_hbm.at[idx_v], rows_v, sem).wait()
        pltpu.sync_copy(rows_v, out_hbm.at[pl.ds(off, CHUNK)])
        return carry

    lax.fori_loop(0, N_CHUNKS, body, 0)


def kernel(token_ids, embedding):
    flat = token_ids.reshape(-1).astype(jnp.int32)
    out = _gather_kernel(flat, embedding)
    return out.reshape(token_ids.shape + (D,))


# speedup vs baseline: 1.0462x; 1.0462x over previous
import functools

import jax
import jax.numpy as jnp
from jax import lax
from jax.experimental import pallas as pl
from jax.experimental.pallas import tpu as pltpu
from jax.experimental.pallas import tpu_sc as plsc

D = 64
B_TOTAL = 4096 * 200

_info = plsc.get_sparse_core_info()
NC, NS = _info.num_cores, _info.num_subcores
NW = NC * NS
B_PER_W = B_TOTAL // NW
CHUNK = 320
N_CHUNKS = B_PER_W // CHUNK
NBUF = 4
S = 2
N_STEPS = N_CHUNKS + S
N_OUTER = (N_STEPS + NBUF - 1) // NBUF

_mesh = plsc.VectorSubcoreMesh(core_axis_name="c", subcore_axis_name="s")


@functools.partial(
    pl.kernel,
    mesh=_mesh,
    out_type=jax.ShapeDtypeStruct((B_TOTAL, D), jnp.float32),
    compiler_params=pltpu.CompilerParams(use_tc_tiling_on_sc=False),
    scratch_types=[
        pltpu.VMEM((B_PER_W,), jnp.int32),
        pltpu.VMEM((CHUNK, D), jnp.float32),
        pltpu.VMEM((CHUNK, D), jnp.float32),
        pltpu.VMEM((CHUNK, D), jnp.float32),
        pltpu.VMEM((CHUNK, D), jnp.float32),
        pltpu.SemaphoreType.DMA,
        pltpu.SemaphoreType.DMA,
        pltpu.SemaphoreType.DMA,
        pltpu.SemaphoreType.DMA,
        pltpu.SemaphoreType.DMA,
        pltpu.SemaphoreType.DMA,
        pltpu.SemaphoreType.DMA,
        pltpu.SemaphoreType.DMA,
    ],
)
def _gather_kernel(idx_hbm, table_hbm, out_hbm, idx_v,
                   rows0, rows1, rows2, rows3,
                   gsem0, gsem1, gsem2, gsem3,
                   wsem0, wsem1, wsem2, wsem3):
    rows = [rows0, rows1, rows2, rows3]
    gsem = [gsem0, gsem1, gsem2, gsem3]
    wsem = [wsem0, wsem1, wsem2, wsem3]

    wid = lax.axis_index("s") * NC + lax.axis_index("c")
    base = wid * B_PER_W

    pltpu.sync_copy(idx_hbm.at[pl.ds(base, B_PER_W)], idx_v)

    def gather_copy(g, b):
        src = table_hbm.at[idx_v.at[pl.ds(g * CHUNK, CHUNK)]]
        return pltpu.make_async_copy(src, rows[b], gsem[b])

    def wb_copy(g, b):
        dst = out_hbm.at[pl.ds(base + g * CHUNK, CHUNK)]
        return pltpu.make_async_copy(rows[b], dst, wsem[b])

    def outer(g0, carry):
        for j in range(NBUF):
            g = g0 * NBUF + j
            bc = (j + NBUF - S) % NBUF

            @pl.when(g >= NBUF)
            def _():
                wb_copy(g - NBUF, j).wait()

            @pl.when(g < N_CHUNKS)
            def _():
                gather_copy(g, j).start()

            @pl.when(jnp.logical_and(g >= S, g < N_CHUNKS + S))
            def _():
                gather_copy(g - S, bc).wait()
                wb_copy(g - S, bc).start()
        return carry

    lax.fori_loop(0, N_OUTER, outer, 0)


def kernel(token_ids, embedding):
    flat = token_ids.reshape(-1).astype(jnp.int32)
    out = _gather_kernel(flat, embedding)
    return out.reshape(token_ids.shape + (D,))
